# baseline (speedup 1.0000x reference)
import functools

import jax
import jax.numpy as jnp
from jax import lax
from jax.experimental import pallas as pl
from jax.experimental.pallas import tpu as pltpu
from jax.experimental.pallas import tpu_sc as plsc

N = 10000
E = 320000
F = 128
H = 16

NC = 2
NS = 16
NW = NC * NS
TE = E // NW
CB = 125
NSUB = TE // CB
BIG = 2000
NBIG = TE // BIG
SPB = BIG // CB
NP = 10240
ROWU = 8
EP = E // 8
EPBLK = 2000

_HIGH = lax.Precision.HIGHEST


def _dot(a, b):
    return jnp.dot(a, b, precision=_HIGH, preferred_element_type=jnp.float32)


def _pq_body(x_ref, w1a_ref, w3a_ref, p_ref, q_ref):
    xv = x_ref[...]
    p_ref[...] = _dot(xv, w1a_ref[...])
    q_ref[...] = _dot(xv, w3a_ref[...])


def _edge_lin_body(ea_ref, w_ref, a_ref, g_ref, cs_ref):
    blk = ea_ref[...]
    a_ref[...] = _dot(blk, w_ref[...])
    gblk = lax.dot_general(blk, blk, (((0,), (0,)), ((), ())),
                           precision=_HIGH,
                           preferred_element_type=jnp.float32)
    csblk = jnp.concatenate(
        [jnp.sum(blk, axis=0, keepdims=True),
         jnp.zeros((7, 128), jnp.float32)], axis=0)

    @pl.when(pl.program_id(0) == 0)
    def _():
        g_ref[...] = jnp.zeros((128, 128), jnp.float32)
        cs_ref[...] = jnp.zeros((8, 128), jnp.float32)

    g_ref[...] += gblk
    cs_ref[...] += csblk


def _stats_body(g_ref, cs_ref, w_ref, sa0_ref, sa1_ref, p_ref, g1_ref,
                bt1_ref, o_ref):
    w = w_ref[...]
    cs = cs_ref[...]
    col_ea = sum(cs[0:1, u * H:(u + 1) * H] for u in range(8))
    col_a = _dot(col_ea, w)
    g16 = sum(g_ref[u * H:(u + 1) * H, u * H:(u + 1) * H] for u in range(8))
    m1 = lax.dot_general(w, g16, (((0,), (0,)), ((), ())),
                         precision=_HIGH, preferred_element_type=jnp.float32)
    m2 = _dot(m1, w)
    col_a2 = jnp.sum(m2 * jnp.eye(H, dtype=jnp.float32), axis=0,
                     keepdims=True)
    sea = sa0_ref[:, 0:H] + sa1_ref[:, 0:H]
    sa = _dot(sea, w)
    outdeg = sa0_ref[:, H:H + 1] + sa1_ref[:, H:H + 1]
    p = p_ref[...]
    sum_b = col_a + jnp.sum(outdeg * p, axis=0, keepdims=True)
    sumsq = (col_a2 + jnp.sum(outdeg * p * p, axis=0, keepdims=True)
             + 2.0 * jnp.sum(p * sa, axis=0, keepdims=True))
    mean_b = sum_b / float(E)
    var = sumsq / float(E) - mean_b * mean_b
    a = g1_ref[...] * lax.rsqrt(var + 1e-5)
    c = bt1_ref[...] - mean_b * a
    o_ref[...] = jnp.concatenate([a, c], axis=0)


def _final_body(q_ref, h0_ref, h1_ref, c0_ref, c1_ref, w2_ref, b2_ref,
                w3b_ref, b3_ref, g2_ref, bt2_ref, w4_ref, b4_ref, o_ref):
    s = h0_ref[...] + h1_ref[...]
    cnt = c0_ref[:, 0:1] + c1_ref[:, 0:1]
    sm = s / jnp.maximum(cnt, 1.0)
    agg = _dot(sm, w2_ref[...]) + b2_ref[...] * (cnt > 0).astype(jnp.float32)
    z1 = q_ref[...] + _dot(agg, w3b_ref[...]) + b3_ref[...]
    m = jnp.mean(z1, axis=0, keepdims=True)
    v = jnp.mean(z1 * z1, axis=0, keepdims=True) - m * m
    zn = jnp.maximum((z1 - m) * lax.rsqrt(v + 1e-5) * g2_ref[...]
                     + bt2_ref[...], 0.0)
    o_ref[...] = _dot(zn, w4_ref[...]) + b4_ref[...]


_MESH = plsc.VectorSubcoreMesh(core_axis_name="c", subcore_axis_name="s")


@functools.partial(
    pl.kernel,
    mesh=_MESH,
    compiler_params=pltpu.CompilerParams(use_tc_tiling_on_sc=False),
    out_type=jax.ShapeDtypeStruct((NC * NP, 2 * H), jnp.float32),
    scratch_types=[
        pltpu.VMEM((NSUB, CB), jnp.int32),
        pltpu.VMEM((BIG // 8, 128), jnp.float32),
        pltpu.VMEM((BIG, 2 * H), jnp.float32),
        pltpu.VMEM_SHARED((NP, 2 * H), jnp.float32),
        pltpu.SemaphoreType.DMA,
    ],
)
def _sc_scatter_a(a_hbm, send_hbm, const_hbm, zeros_hbm, out_hbm,
                  idx_v, abuf, sbuf, acc, dsem):
    cid = lax.axis_index("c")
    sid = lax.axis_index("s")
    wid = sid * NC + cid
    basep = wid * (TE // 8)

    @pl.when(sid == 0)
    def _():
        pltpu.sync_copy(zeros_hbm, acc)

    pltpu.sync_copy(const_hbm, sbuf)
    pltpu.sync_copy(send_hbm.at[wid], idx_v)
    plsc.subcore_barrier()

    def big_body(b, carry):
        rowp = basep + b * (BIG // 8)
        pltpu.async_copy(a_hbm.at[pl.ds(rowp, BIG // 8)], abuf, dsem).wait()

        def row_body(r0, carry2):
            for u in range(ROWU):
                sbuf[r0 * ROWU + u, 0:H] = abuf[r0, u * H:(u + 1) * H]
            return carry2

        lax.fori_loop(0, BIG // ROWU, row_body, 0, unroll=False)
        for j in range(SPB):
            pltpu.sync_copy(sbuf.at[pl.ds(j * CB, CB)],
                            acc.at[idx_v.at[b * SPB + j]], add=True)
        return carry

    lax.fori_loop(0, NBIG, big_body, 0, unroll=False)
    plsc.subcore_barrier()
    rows = NP // NS
    pltpu.sync_copy(acc.at[pl.ds(sid * rows, rows)],
                    out_hbm.at[pl.ds(cid * NP + sid * rows, rows)])


@functools.partial(
    pl.kernel,
    mesh=_MESH,
    compiler_params=pltpu.CompilerParams(use_tc_tiling_on_sc=False),
    out_type=[
        jax.ShapeDtypeStruct((NC * NP, H), jnp.float32),
        jax.ShapeDtypeStruct((NC * NP, H), jnp.float32),
    ],
    scratch_types=[
        pltpu.VMEM((NSUB, CB), jnp.int32),
        pltpu.VMEM((NSUB, CB), jnp.int32),
        pltpu.VMEM((BIG, H), jnp.float32),
        pltpu.VMEM((BIG // 8, 128), jnp.float32),
        pltpu.VMEM((CB, H), jnp.float32),
        pltpu.VMEM((2, H), jnp.float32),
        pltpu.VMEM_SHARED((NP, H), jnp.float32),
        pltpu.VMEM_SHARED((NP, H), jnp.float32),
        pltpu.SemaphoreType.DMA,
        pltpu.SemaphoreType.DMA,
    ],
)
def _sc_edge(p_hbm, a_hbm, send_hbm, rec_hbm, ac_hbm, cnt1_hbm, zeros_hbm,
             outh_hbm, outc_hbm, sidx_v, ridx_v, gbuf, abuf, cbuf, acv,
             acc_h, acc_c, gsem, dsem):
    cid = lax.axis_index("c")
    sid = lax.axis_index("s")
    wid = sid * NC + cid
    base = wid * TE
    basep = wid * (TE // 8)

    @pl.when(sid == 0)
    def _():
        pltpu.sync_copy(zeros_hbm, acc_h)
        pltpu.sync_copy(zeros_hbm, acc_c)

    pltpu.sync_copy(cnt1_hbm, cbuf)
    pltpu.sync_copy(ac_hbm, acv)
    pltpu.sync_copy(send_hbm.at[wid], sidx_v)
    pltpu.sync_copy(rec_hbm.at[wid], ridx_v)
    plsc.subcore_barrier()

    av = acv[0]
    cv = acv[1]

    def big_body(b, carry):
        rowp = basep + b * (BIG // 8)
        a_cp = pltpu.async_copy(a_hbm.at[pl.ds(rowp, BIG // 8)], abuf, dsem)
        gathers = []
        for j in range(SPB):
            gathers.append(pltpu.async_copy(
                p_hbm.at[sidx_v.at[b * SPB + j]],
                gbuf.at[pl.ds(j * CB, CB)], gsem))
        a_cp.wait()
        for g in gathers:
            g.wait()

        def row_body(r0, carry2):
            for u in range(ROWU):
                r = r0 * ROWU + u
                hv = gbuf[r] + abuf[r0, u * H:(u + 1) * H]
                gbuf[r] = jnp.maximum(hv * av + cv, 0.0)
            return carry2

        lax.fori_loop(0, BIG // ROWU, row_body, 0, unroll=False)
        for j in range(SPB):
            idx = ridx_v.at[b * SPB + j]
            pltpu.sync_copy(gbuf.at[pl.ds(j * CB, CB)],
                            acc_h.at[idx], add=True)
            pltpu.sync_copy(cbuf, acc_c.at[idx], add=True)
        return carry

    lax.fori_loop(0, NBIG, big_body, 0, unroll=False)
    plsc.subcore_barrier()
    rows = NP // NS
    pltpu.sync_copy(acc_h.at[pl.ds(sid * rows, rows)],
                    outh_hbm.at[pl.ds(cid * NP + sid * rows, rows)])
    pltpu.sync_copy(acc_c.at[pl.ds(sid * rows, rows)],
                    outc_hbm.at[pl.ds(cid * NP + sid * rows, rows)])


def kernel(x, edge_index, edge_attr, u, batch, W1, b1, g1, bt1, W2, b2,
           W3, b3, g2, bt2, W4, b4):
    del u, batch
    send = edge_index[0].astype(jnp.int32).reshape(NW, NSUB, CB)
    rec = edge_index[1].astype(jnp.int32).reshape(NW, NSUB, CB)
    eap = edge_attr.reshape(EP, 128)
    wd = jnp.kron(jnp.eye(8, dtype=jnp.float32), W1[F:])

    p, q = pl.pallas_call(
        _pq_body,
        out_shape=[jax.ShapeDtypeStruct((N, H), jnp.float32),
                   jax.ShapeDtypeStruct((N, H), jnp.float32)],
    )(x, W1[:F], W3[:F])

    nblk = EP // EPBLK
    a_mat, gacc, csums = pl.pallas_call(
        _edge_lin_body,
        grid=(nblk,),
        in_specs=[pl.BlockSpec((EPBLK, 128), lambda i: (i, 0)),
                  pl.BlockSpec((128, 128), lambda i: (0, 0))],
        out_specs=[pl.BlockSpec((EPBLK, 128), lambda i: (i, 0)),
                   pl.BlockSpec((128, 128), lambda i: (0, 0)),
                   pl.BlockSpec((8, 128), lambda i: (0, 0))],
        out_shape=[jax.ShapeDtypeStruct((EP, 128), jnp.float32),
                   jax.ShapeDtypeStruct((128, 128), jnp.float32),
                   jax.ShapeDtypeStruct((8, 128), jnp.float32)],
    )(eap, wd)

    const = jnp.zeros((BIG, 2 * H), jnp.float32).at[:, H].set(1.0)
    cnt1 = jnp.zeros((CB, H), jnp.float32).at[:, 0].set(1.0)
    zeros32 = jnp.zeros((NP, 2 * H), jnp.float32)
    zeros16 = jnp.zeros((NP, H), jnp.float32)

    sa_acc = _sc_scatter_a(eap, send, const, zeros32)
    sa_acc = sa_acc.reshape(NC, NP, 2 * H)[:, :N, :]

    ac = pl.pallas_call(
        _stats_body,
        out_shape=jax.ShapeDtypeStruct((2, H), jnp.float32),
    )(gacc, csums, W1[F:], sa_acc[0], sa_acc[1], p, g1[None], bt1[None])

    acc_h, acc_c = _sc_edge(p, a_mat, send, rec, ac, cnt1, zeros16)
    acc_h = acc_h.reshape(NC, NP, H)[:, :N, :]
    acc_c = acc_c.reshape(NC, NP, H)[:, :N, :]

    z = pl.pallas_call(
        _final_body,
        out_shape=jax.ShapeDtypeStruct((N, H), jnp.float32),
    )(q, acc_h[0], acc_h[1], acc_c[0], acc_c[1], W2, b2[None], W3[F:],
      b3[None], g2[None], bt2[None], W4, b4[None])
    return z


# speedup vs baseline: 1.3133x; 1.3133x over previous
import functools

import jax
import jax.numpy as jnp
from jax import lax
from jax.experimental import pallas as pl
from jax.experimental.pallas import tpu as pltpu
from jax.experimental.pallas import tpu_sc as plsc

N = 10000
E = 320000
F = 128
H = 16

NC = 2
NS = 16
NW = NC * NS
TE = E // NW
CB = 125
NSUB = TE // CB
BIG = 2000
NBIG = TE // BIG
SPB = BIG // CB
NP = 10240
ROWU = 8
EP = E // 8
EPBLK = 2000

_HIGH = lax.Precision.HIGHEST


def _dot(a, b):
    return jnp.dot(a, b, precision=_HIGH, preferred_element_type=jnp.float32)


def _pq_body(x_ref, w1a_ref, w3a_ref, p_ref, q_ref):
    xv = x_ref[...]
    p_ref[...] = _dot(xv, w1a_ref[...])
    q_ref[...] = _dot(xv, w3a_ref[...])


def _edge_lin_body(ea_ref, w_ref, a_ref, el_ref, g_ref, cs_ref):
    blk = ea_ref[...]
    a_ref[...] = _dot(blk, w_ref[...])
    el_ref[...] = blk
    gblk = lax.dot_general(blk, blk, (((0,), (0,)), ((), ())),
                           precision=_HIGH,
                           preferred_element_type=jnp.float32)
    csblk = jnp.concatenate(
        [jnp.sum(blk, axis=0, keepdims=True),
         jnp.zeros((7, 128), jnp.float32)], axis=0)

    @pl.when(pl.program_id(0) == 0)
    def _():
        g_ref[...] = jnp.zeros((128, 128), jnp.float32)
        cs_ref[...] = jnp.zeros((8, 128), jnp.float32)

    g_ref[...] += gblk
    cs_ref[...] += csblk


def _stats_body(g_ref, cs_ref, w_ref, sea0_ref, sea1_ref, od0_ref, od1_ref,
                p_ref, g1_ref, bt1_ref, o_ref):
    w = w_ref[...]
    cs = cs_ref[...]
    col_ea = sum(cs[0:1, u * H:(u + 1) * H] for u in range(8))
    col_a = _dot(col_ea, w)
    g16 = sum(g_ref[u * H:(u + 1) * H, u * H:(u + 1) * H] for u in range(8))
    m1 = lax.dot_general(w, g16, (((0,), (0,)), ((), ())),
                         precision=_HIGH, preferred_element_type=jnp.float32)
    m2 = _dot(m1, w)
    col_a2 = jnp.sum(m2 * jnp.eye(H, dtype=jnp.float32), axis=0,
                     keepdims=True)
    sea = sea0_ref[...] + sea1_ref[...]
    sa = _dot(sea, w)
    outdeg = od0_ref[:, 0:1] + od1_ref[:, 0:1]
    p = p_ref[...]
    sum_b = col_a + jnp.sum(outdeg * p, axis=0, keepdims=True)
    sumsq = (col_a2 + jnp.sum(outdeg * p * p, axis=0, keepdims=True)
             + 2.0 * jnp.sum(p * sa, axis=0, keepdims=True))
    mean_b = sum_b / float(E)
    var = sumsq / float(E) - mean_b * mean_b
    a = g1_ref[...] * lax.rsqrt(var + 1e-5)
    c = bt1_ref[...] - mean_b * a
    o_ref[...] = jnp.concatenate([a, c], axis=0)


def _final_body(q_ref, h0_ref, h1_ref, c0_ref, c1_ref, w2_ref, b2_ref,
                w3b_ref, b3_ref, g2_ref, bt2_ref, w4_ref, b4_ref, o_ref):
    s = h0_ref[...] + h1_ref[...]
    cnt = c0_ref[:, 0:1] + c1_ref[:, 0:1]
    sm = s / jnp.maximum(cnt, 1.0)
    agg = _dot(sm, w2_ref[...]) + b2_ref[...] * (cnt > 0).astype(jnp.float32)
    z1 = q_ref[...] + _dot(agg, w3b_ref[...]) + b3_ref[...]
    m = jnp.mean(z1, axis=0, keepdims=True)
    v = jnp.mean(z1 * z1, axis=0, keepdims=True) - m * m
    zn = jnp.maximum((z1 - m) * lax.rsqrt(v + 1e-5) * g2_ref[...]
                     + bt2_ref[...], 0.0)
    o_ref[...] = _dot(zn, w4_ref[...]) + b4_ref[...]


_MESH = plsc.VectorSubcoreMesh(core_axis_name="c", subcore_axis_name="s")


@functools.partial(
    pl.kernel,
    mesh=_MESH,
    compiler_params=pltpu.CompilerParams(use_tc_tiling_on_sc=False),
    out_type=[
        jax.ShapeDtypeStruct((NC * NP, H), jnp.float32),
        jax.ShapeDtypeStruct((NC * NP, H), jnp.float32),
    ],
    scratch_types=[
        pltpu.VMEM((NSUB, CB), jnp.int32),
        pltpu.VMEM((NSUB, CB), jnp.int32),
        pltpu.VMEM((CB, H), jnp.float32),
        pltpu.VMEM_SHARED((NP, H), jnp.float32),
        pltpu.VMEM_SHARED((NP, H), jnp.float32),
    ],
)
def _sc_hist(send_hbm, rec_hbm, cnt1_hbm, zeros_hbm, outs_hbm, outr_hbm,
             sidx_v, ridx_v, cbuf, acc_s, acc_r):
    cid = lax.axis_index("c")
    sid = lax.axis_index("s")
    wid = sid * NC + cid

    @pl.when(sid == 0)
    def _():
        pltpu.sync_copy(zeros_hbm, acc_s)
        pltpu.sync_copy(zeros_hbm, acc_r)

    pltpu.sync_copy(cnt1_hbm, cbuf)
    pltpu.sync_copy(send_hbm.at[wid], sidx_v)
    pltpu.sync_copy(rec_hbm.at[wid], ridx_v)
    plsc.subcore_barrier()

    def body(j, carry):
        pltpu.sync_copy(cbuf, acc_s.at[sidx_v.at[j]], add=True)
        pltpu.sync_copy(cbuf, acc_r.at[ridx_v.at[j]], add=True)
        return carry

    lax.fori_loop(0, NSUB, body, 0, unroll=False)
    plsc.subcore_barrier()
    rows = NP // NS
    pltpu.sync_copy(acc_s.at[pl.ds(sid * rows, rows)],
                    outs_hbm.at[pl.ds(cid * NP + sid * rows, rows)])
    pltpu.sync_copy(acc_r.at[pl.ds(sid * rows, rows)],
                    outr_hbm.at[pl.ds(cid * NP + sid * rows, rows)])


@functools.partial(
    pl.kernel,
    mesh=_MESH,
    compiler_params=pltpu.CompilerParams(use_tc_tiling_on_sc=False),
    out_type=jax.ShapeDtypeStruct((NC * NP, H), jnp.float32),
    scratch_types=[
        pltpu.VMEM((NSUB, CB), jnp.int32),
        pltpu.VMEM((BIG, H), jnp.float32),
        pltpu.VMEM_SHARED((NP, H), jnp.float32),
        pltpu.SemaphoreType.DMA,
    ],
)
def _sc_sea(ea_hbm, send_hbm, zeros_hbm, out_hbm, idx_v, abuf, acc, dsem):
    cid = lax.axis_index("c")
    sid = lax.axis_index("s")
    wid = sid * NC + cid
    base = wid * TE

    @pl.when(sid == 0)
    def _():
        pltpu.sync_copy(zeros_hbm, acc)

    pltpu.sync_copy(send_hbm.at[wid], idx_v)
    plsc.subcore_barrier()

    def big_body(b, carry):
        row0 = base + b * BIG
        pltpu.async_copy(ea_hbm.at[pl.ds(row0, BIG)], abuf, dsem).wait()
        for j in range(SPB):
            pltpu.sync_copy(abuf.at[pl.ds(j * CB, CB)],
                            acc.at[idx_v.at[b * SPB + j]], add=True)
        return carry

    lax.fori_loop(0, NBIG, big_body, 0, unroll=False)
    plsc.subcore_barrier()
    rows = NP // NS
    pltpu.sync_copy(acc.at[pl.ds(sid * rows, rows)],
                    out_hbm.at[pl.ds(cid * NP + sid * rows, rows)])


@functools.partial(
    pl.kernel,
    mesh=_MESH,
    compiler_params=pltpu.CompilerParams(use_tc_tiling_on_sc=False),
    out_type=jax.ShapeDtypeStruct((NC * NP, H), jnp.float32),
    scratch_types=[
        pltpu.VMEM((NSUB, CB), jnp.int32),
        pltpu.VMEM((NSUB, CB), jnp.int32),
        pltpu.VMEM((BIG, H), jnp.float32),
        pltpu.VMEM((BIG // 8, 128), jnp.float32),
        pltpu.VMEM((2, H), jnp.float32),
        pltpu.VMEM_SHARED((NP, H), jnp.float32),
        pltpu.SemaphoreType.DMA,
        pltpu.SemaphoreType.DMA,
    ],
)
def _sc_edge(p_hbm, a_hbm, send_hbm, rec_hbm, ac_hbm, zeros_hbm,
             outh_hbm, sidx_v, ridx_v, gbuf, abuf, acv, acc_h, gsem, dsem):
    cid = lax.axis_index("c")
    sid = lax.axis_index("s")
    wid = sid * NC + cid
    basep = wid * (TE // 8)

    @pl.when(sid == 0)
    def _():
        pltpu.sync_copy(zeros_hbm, acc_h)

    pltpu.sync_copy(ac_hbm, acv)
    pltpu.sync_copy(send_hbm.at[wid], sidx_v)
    pltpu.sync_copy(rec_hbm.at[wid], ridx_v)
    plsc.subcore_barrier()

    av = acv[0]
    cv = acv[1]

    def big_body(b, carry):
        rowp = basep + b * (BIG // 8)
        a_cp = pltpu.async_copy(a_hbm.at[pl.ds(rowp, BIG // 8)], abuf, dsem)
        gathers = []
        for j in range(SPB):
            gathers.append(pltpu.async_copy(
                p_hbm.at[sidx_v.at[b * SPB + j]],
                gbuf.at[pl.ds(j * CB, CB)], gsem))
        a_cp.wait()
        for g in gathers:
            g.wait()

        def row_body(r0, carry2):
            for u in range(ROWU):
                r = r0 * ROWU + u
                hv = gbuf[r] + abuf[r0, u * H:(u + 1) * H]
                gbuf[r] = jnp.maximum(hv * av + cv, 0.0)
            return carry2

        lax.fori_loop(0, BIG // ROWU, row_body, 0, unroll=False)
        for j in range(SPB):
            pltpu.sync_copy(gbuf.at[pl.ds(j * CB, CB)],
                            acc_h.at[ridx_v.at[b * SPB + j]], add=True)
        return carry

    lax.fori_loop(0, NBIG, big_body, 0, unroll=False)
    plsc.subcore_barrier()
    rows = NP // NS
    pltpu.sync_copy(acc_h.at[pl.ds(sid * rows, rows)],
                    outh_hbm.at[pl.ds(cid * NP + sid * rows, rows)])


def kernel(x, edge_index, edge_attr, u, batch, W1, b1, g1, bt1, W2, b2,
           W3, b3, g2, bt2, W4, b4):
    del u, batch
    send = edge_index[0].astype(jnp.int32).reshape(NW, NSUB, CB)
    rec = edge_index[1].astype(jnp.int32).reshape(NW, NSUB, CB)
    eap = edge_attr.reshape(EP, 128)
    wd = jnp.kron(jnp.eye(8, dtype=jnp.float32), W1[F:])

    cnt1 = jnp.zeros((CB, H), jnp.float32).at[:, 0].set(1.0)
    zeros16 = jnp.zeros((NP, H), jnp.float32)

    od_acc, cn_acc = _sc_hist(send, rec, cnt1, zeros16)
    od_acc = od_acc.reshape(NC, NP, H)
    cn_acc = cn_acc.reshape(NC, NP, H)[:, :N, :]

    p, q = pl.pallas_call(
        _pq_body,
        out_shape=[jax.ShapeDtypeStruct((N, H), jnp.float32),
                   jax.ShapeDtypeStruct((N, H), jnp.float32)],
    )(x, W1[:F], W3[:F])

    nblk = EP // EPBLK
    a_mat, ea_lin, gacc, csums = pl.pallas_call(
        _edge_lin_body,
        grid=(nblk,),
        in_specs=[pl.BlockSpec((EPBLK, 128), lambda i: (i, 0)),
                  pl.BlockSpec((128, 128), lambda i: (0, 0))],
        out_specs=[pl.BlockSpec((EPBLK, 128), lambda i: (i, 0)),
                   pl.BlockSpec((EPBLK, 128), lambda i: (i, 0)),
                   pl.BlockSpec((128, 128), lambda i: (0, 0)),
                   pl.BlockSpec((8, 128), lambda i: (0, 0))],
        out_shape=[jax.ShapeDtypeStruct((EP, 128), jnp.float32),
                   jax.ShapeDtypeStruct((EP, 128), jnp.float32),
                   jax.ShapeDtypeStruct((128, 128), jnp.float32),
                   jax.ShapeDtypeStruct((8, 128), jnp.float32)],
    )(eap, wd)

    sea_acc = _sc_sea(ea_lin.reshape(E, H), send, zeros16)
    sea_acc = sea_acc.reshape(NC, NP, H)[:, :N, :]

    ac = pl.pallas_call(
        _stats_body,
        out_shape=jax.ShapeDtypeStruct((2, H), jnp.float32),
    )(gacc, csums, W1[F:], sea_acc[0], sea_acc[1],
      od_acc[0, :N], od_acc[1, :N], p, g1[None], bt1[None])

    acc_h = _sc_edge(p, a_mat, send, rec, ac, zeros16)
    acc_h = acc_h.reshape(NC, NP, H)[:, :N, :]

    z = pl.pallas_call(
        _final_body,
        out_shape=jax.ShapeDtypeStruct((N, H), jnp.float32),
    )(q, acc_h[0], acc_h[1], cn_acc[0], cn_acc[1], W2, b2[None], W3[F:],
      b3[None], g2[None], bt2[None], W4, b4[None])
    return z


# speedup vs baseline: 1.4030x; 1.0683x over previous
import functools

import jax
import jax.numpy as jnp
from jax import lax
from jax.experimental import pallas as pl
from jax.experimental.pallas import tpu as pltpu
from jax.experimental.pallas import tpu_sc as plsc

N = 10000
E = 320000
F = 128
H = 16

NC = 2
NS = 16
NW = NC * NS
TE = E // NW
CB = 125
NSUB = TE // CB
BIG = 2000
NBIG = TE // BIG
SPB = BIG // CB
NP = 10240
ROWU = 8
EP = E // 8
EPBLK = 2000

_HIGH = lax.Precision.HIGHEST


def _dot(a, b):
    return jnp.dot(a, b, precision=_HIGH, preferred_element_type=jnp.float32)


def _pq_body(x_ref, w1a_ref, w3a_ref, p_ref, q_ref):
    xv = x_ref[...]
    p_ref[...] = jnp.concatenate(
        [_dot(xv, w1a_ref[...]), jnp.zeros((NP - N, H), jnp.float32)], axis=0)
    q_ref[...] = _dot(xv, w3a_ref[...])


def _edge_lin_body(ea_ref, w_ref, a_ref, el_ref, g_ref, cs_ref):
    blk = ea_ref[...]
    a_ref[...] = _dot(blk, w_ref[...])
    el_ref[...] = blk
    gblk = lax.dot_general(blk, blk, (((0,), (0,)), ((), ())),
                           precision=_HIGH,
                           preferred_element_type=jnp.float32)
    csblk = jnp.concatenate(
        [jnp.sum(blk, axis=0, keepdims=True),
         jnp.zeros((7, 128), jnp.float32)], axis=0)

    @pl.when(pl.program_id(0) == 0)
    def _():
        g_ref[...] = jnp.zeros((128, 128), jnp.float32)
        cs_ref[...] = jnp.zeros((8, 128), jnp.float32)

    g_ref[...] += gblk
    cs_ref[...] += csblk


def _stats_body(g_ref, cs_ref, w_ref, sea_ref, od_ref, p_ref, g1_ref,
                bt1_ref, o_ref):
    w = w_ref[...]
    cs = cs_ref[...]
    col_ea = sum(cs[0:1, u * H:(u + 1) * H] for u in range(8))
    col_a = _dot(col_ea, w)
    g16 = sum(g_ref[u * H:(u + 1) * H, u * H:(u + 1) * H] for u in range(8))
    m1 = lax.dot_general(w, g16, (((0,), (0,)), ((), ())),
                         precision=_HIGH, preferred_element_type=jnp.float32)
    m2 = _dot(m1, w)
    col_a2 = jnp.sum(m2 * jnp.eye(H, dtype=jnp.float32), axis=0,
                     keepdims=True)
    sea = sea_ref[0:NP] + sea_ref[NP:2 * NP]
    sa = _dot(sea, w)
    outdeg = od_ref[0:NP, 0:1] + od_ref[NP:2 * NP, 0:1]
    p = p_ref[...]
    sum_b = col_a + jnp.sum(outdeg * p, axis=0, keepdims=True)
    sumsq = (col_a2 + jnp.sum(outdeg * p * p, axis=0, keepdims=True)
             + 2.0 * jnp.sum(p * sa, axis=0, keepdims=True))
    mean_b = sum_b / float(E)
    var = sumsq / float(E) - mean_b * mean_b
    a = g1_ref[...] * lax.rsqrt(var + 1e-5)
    c = bt1_ref[...] - mean_b * a
    o_ref[...] = jnp.concatenate([a, c], axis=0)


def _final_body(q_ref, hr_ref, cr_ref, w2_ref, b2_ref,
                w3b_ref, b3_ref, g2_ref, bt2_ref, w4_ref, b4_ref, o_ref):
    s = hr_ref[0:N] + hr_ref[NP:NP + N]
    cnt = cr_ref[0:N, 0:1] + cr_ref[NP:NP + N, 0:1]
    sm = s / jnp.maximum(cnt, 1.0)
    agg = _dot(sm, w2_ref[...]) + b2_ref[...] * (cnt > 0).astype(jnp.float32)
    z1 = q_ref[...] + _dot(agg, w3b_ref[...]) + b3_ref[...]
    m = jnp.mean(z1, axis=0, keepdims=True)
    v = jnp.mean(z1 * z1, axis=0, keepdims=True) - m * m
    zn = jnp.maximum((z1 - m) * lax.rsqrt(v + 1e-5) * g2_ref[...]
                     + bt2_ref[...], 0.0)
    o_ref[...] = _dot(zn, w4_ref[...]) + b4_ref[...]


_MESH = plsc.VectorSubcoreMesh(core_axis_name="c", subcore_axis_name="s")


@functools.partial(
    pl.kernel,
    mesh=_MESH,
    compiler_params=pltpu.CompilerParams(use_tc_tiling_on_sc=False),
    out_type=[
        jax.ShapeDtypeStruct((NC * NP, H), jnp.float32),
        jax.ShapeDtypeStruct((NC * NP, H), jnp.float32),
    ],
    scratch_types=[
        pltpu.VMEM((NSUB, CB), jnp.int32),
        pltpu.VMEM((NSUB, CB), jnp.int32),
        pltpu.VMEM((CB, H), jnp.float32),
        pltpu.VMEM_SHARED((NP, H), jnp.float32),
        pltpu.VMEM_SHARED((NP, H), jnp.float32),
    ],
)
def _sc_hist(send_hbm, rec_hbm, cnt1_hbm, zeros_hbm, outs_hbm, outr_hbm,
             sidx_v, ridx_v, cbuf, acc_s, acc_r):
    cid = lax.axis_index("c")
    sid = lax.axis_index("s")
    wid = sid * NC + cid

    @pl.when(sid == 0)
    def _():
        pltpu.sync_copy(zeros_hbm, acc_s)
        pltpu.sync_copy(zeros_hbm, acc_r)

    pltpu.sync_copy(cnt1_hbm, cbuf)
    pltpu.sync_copy(send_hbm.at[wid], sidx_v)
    pltpu.sync_copy(rec_hbm.at[wid], ridx_v)
    plsc.subcore_barrier()

    def body(j, carry):
        pltpu.sync_copy(cbuf, acc_s.at[sidx_v.at[j]], add=True)
        pltpu.sync_copy(cbuf, acc_r.at[ridx_v.at[j]], add=True)
        return carry

    lax.fori_loop(0, NSUB, body, 0, unroll=False)
    plsc.subcore_barrier()
    rows = NP // NS
    pltpu.sync_copy(acc_s.at[pl.ds(sid * rows, rows)],
                    outs_hbm.at[pl.ds(cid * NP + sid * rows, rows)])
    pltpu.sync_copy(acc_r.at[pl.ds(sid * rows, rows)],
                    outr_hbm.at[pl.ds(cid * NP + sid * rows, rows)])


@functools.partial(
    pl.kernel,
    mesh=_MESH,
    compiler_params=pltpu.CompilerParams(use_tc_tiling_on_sc=False),
    out_type=jax.ShapeDtypeStruct((NC * NP, H), jnp.float32),
    scratch_types=[
        pltpu.VMEM((NSUB, CB), jnp.int32),
        pltpu.VMEM((BIG, H), jnp.float32),
        pltpu.VMEM_SHARED((NP, H), jnp.float32),
        pltpu.SemaphoreType.DMA,
    ],
)
def _sc_sea(ea_hbm, send_hbm, zeros_hbm, out_hbm, idx_v, abuf, acc, dsem):
    cid = lax.axis_index("c")
    sid = lax.axis_index("s")
    wid = sid * NC + cid
    base = wid * TE

    @pl.when(sid == 0)
    def _():
        pltpu.sync_copy(zeros_hbm, acc)

    pltpu.sync_copy(send_hbm.at[wid], idx_v)
    plsc.subcore_barrier()

    def big_body(b, carry):
        row0 = base + b * BIG
        pltpu.async_copy(ea_hbm.at[pl.ds(row0, BIG)], abuf, dsem).wait()
        for j in range(SPB):
            pltpu.sync_copy(abuf.at[pl.ds(j * CB, CB)],
                            acc.at[idx_v.at[b * SPB + j]], add=True)
        return carry

    lax.fori_loop(0, NBIG, big_body, 0, unroll=False)
    plsc.subcore_barrier()
    rows = NP // NS
    pltpu.sync_copy(acc.at[pl.ds(sid * rows, rows)],
                    out_hbm.at[pl.ds(cid * NP + sid * rows, rows)])


@functools.partial(
    pl.kernel,
    mesh=_MESH,
    compiler_params=pltpu.CompilerParams(use_tc_tiling_on_sc=False),
    out_type=jax.ShapeDtypeStruct((NC * NP, H), jnp.float32),
    scratch_types=[
        pltpu.VMEM((NSUB, CB), jnp.int32),
        pltpu.VMEM((NSUB, CB), jnp.int32),
        pltpu.VMEM((BIG, H), jnp.float32),
        pltpu.VMEM((BIG // 8, 128), jnp.float32),
        pltpu.VMEM((2, H), jnp.float32),
        pltpu.VMEM_SHARED((NP, H), jnp.float32),
        pltpu.SemaphoreType.DMA,
        pltpu.SemaphoreType.DMA,
    ],
)
def _sc_edge(p_hbm, a_hbm, send_hbm, rec_hbm, ac_hbm, zeros_hbm,
             outh_hbm, sidx_v, ridx_v, gbuf, abuf, acv, acc_h, gsem, dsem):
    cid = lax.axis_index("c")
    sid = lax.axis_index("s")
    wid = sid * NC + cid
    basep = wid * (TE // 8)

    @pl.when(sid == 0)
    def _():
        pltpu.sync_copy(zeros_hbm, acc_h)

    pltpu.sync_copy(ac_hbm, acv)
    pltpu.sync_copy(send_hbm.at[wid], sidx_v)
    pltpu.sync_copy(rec_hbm.at[wid], ridx_v)
    plsc.subcore_barrier()

    av = acv[0]
    cv = acv[1]

    def big_body(b, carry):
        rowp = basep + b * (BIG // 8)
        a_cp = pltpu.async_copy(a_hbm.at[pl.ds(rowp, BIG // 8)], abuf, dsem)
        gathers = []
        for j in range(SPB):
            gathers.append(pltpu.async_copy(
                p_hbm.at[sidx_v.at[b * SPB + j]],
                gbuf.at[pl.ds(j * CB, CB)], gsem))
        a_cp.wait()
        for g in gathers:
            g.wait()

        def row_body(r0, carry2):
            for u in range(ROWU):
                r = r0 * ROWU + u
                hv = gbuf[r] + abuf[r0, u * H:(u + 1) * H]
                gbuf[r] = jnp.maximum(hv * av + cv, 0.0)
            return carry2

        lax.fori_loop(0, BIG // ROWU, row_body, 0, unroll=False)
        for j in range(SPB):
            pltpu.sync_copy(gbuf.at[pl.ds(j * CB, CB)],
                            acc_h.at[ridx_v.at[b * SPB + j]], add=True)
        return carry

    lax.fori_loop(0, NBIG, big_body, 0, unroll=False)
    plsc.subcore_barrier()
    rows = NP // NS
    pltpu.sync_copy(acc_h.at[pl.ds(sid * rows, rows)],
                    outh_hbm.at[pl.ds(cid * NP + sid * rows, rows)])


def kernel(x, edge_index, edge_attr, u, batch, W1, b1, g1, bt1, W2, b2,
           W3, b3, g2, bt2, W4, b4):
    del u, batch
    send = edge_index[0].astype(jnp.int32).reshape(NW, NSUB, CB)
    rec = edge_index[1].astype(jnp.int32).reshape(NW, NSUB, CB)
    eap = edge_attr.reshape(EP, 128)
    wd = jnp.kron(jnp.eye(8, dtype=jnp.float32), W1[F:])

    cnt1 = jnp.zeros((CB, H), jnp.float32).at[:, 0].set(1.0)
    zeros16 = jnp.zeros((NP, H), jnp.float32)

    od_acc, cn_acc = _sc_hist(send, rec, cnt1, zeros16)

    p, q = pl.pallas_call(
        _pq_body,
        out_shape=[jax.ShapeDtypeStruct((NP, H), jnp.float32),
                   jax.ShapeDtypeStruct((N, H), jnp.float32)],
    )(x, W1[:F], W3[:F])

    nblk = EP // EPBLK
    a_mat, ea_lin, gacc, csums = pl.pallas_call(
        _edge_lin_body,
        grid=(nblk,),
        in_specs=[pl.BlockSpec((EPBLK, 128), lambda i: (i, 0)),
                  pl.BlockSpec((128, 128), lambda i: (0, 0))],
        out_specs=[pl.BlockSpec((EPBLK, 128), lambda i: (i, 0)),
                   pl.BlockSpec((EPBLK, 128), lambda i: (i, 0)),
                   pl.BlockSpec((128, 128), lambda i: (0, 0)),
                   pl.BlockSpec((8, 128), lambda i: (0, 0))],
        out_shape=[jax.ShapeDtypeStruct((EP, 128), jnp.float32),
                   jax.ShapeDtypeStruct((EP, 128), jnp.float32),
                   jax.ShapeDtypeStruct((128, 128), jnp.float32),
                   jax.ShapeDtypeStruct((8, 128), jnp.float32)],
    )(eap, wd)

    sea_acc = _sc_sea(ea_lin.reshape(E, H), send, zeros16)

    ac = pl.pallas_call(
        _stats_body,
        out_shape=jax.ShapeDtypeStruct((2, H), jnp.float32),
    )(gacc, csums, W1[F:], sea_acc, od_acc, p, g1[None], bt1[None])

    acc_h = _sc_edge(p, a_mat, send, rec, ac, zeros16)

    z = pl.pallas_call(
        _final_body,
        out_shape=jax.ShapeDtypeStruct((N, H), jnp.float32),
    )(q, acc_h, cn_acc, W2, b2[None], W3[F:],
      b3[None], g2[None], bt2[None], W4, b4[None])
    return z


# speedup vs baseline: 1.5326x; 1.0924x over previous
import functools

import jax
import jax.numpy as jnp
from jax import lax
from jax.experimental import pallas as pl
from jax.experimental.pallas import tpu as pltpu
from jax.experimental.pallas import tpu_sc as plsc

N = 10000
E = 320000
F = 128
H = 16

NC = 2
NS = 16
NW = NC * NS
TE = E // NW
CB = 125
NSUB = TE // CB
BIG = 2000
NBIG = TE // BIG
SPB = BIG // CB
NP = 10240
ROWU = 8
EP = E // 8
EPBLK = 2000

_HIGH = lax.Precision.HIGHEST


def _dot(a, b):
    return jnp.dot(a, b, precision=_HIGH, preferred_element_type=jnp.float32)


def _pq_body(x_ref, w1a_ref, w3a_ref, p_ref, q_ref):
    xv = x_ref[...]
    p_ref[...] = jnp.concatenate(
        [_dot(xv, w1a_ref[...]), jnp.zeros((NP - N, H), jnp.float32)], axis=0)
    q_ref[...] = _dot(xv, w3a_ref[...])


def _edge_lin_body(ea_ref, w_ref, a_ref, el_ref, g_ref, cs_ref):
    blk = ea_ref[...]
    a_ref[...] = _dot(blk, w_ref[...])
    el_ref[...] = blk
    gblk = lax.dot_general(blk, blk, (((0,), (0,)), ((), ())),
                           precision=_HIGH,
                           preferred_element_type=jnp.float32)
    csblk = jnp.concatenate(
        [jnp.sum(blk, axis=0, keepdims=True),
         jnp.zeros((7, 128), jnp.float32)], axis=0)

    @pl.when(pl.program_id(0) == 0)
    def _():
        g_ref[...] = jnp.zeros((128, 128), jnp.float32)
        cs_ref[...] = jnp.zeros((8, 128), jnp.float32)

    g_ref[...] += gblk
    cs_ref[...] += csblk


def _stats_body(g_ref, cs_ref, w_ref, sea_ref, od_ref, p_ref, g1_ref,
                bt1_ref, o_ref):
    w = w_ref[...]
    cs = cs_ref[...]
    col_ea = sum(cs[0:1, u * H:(u + 1) * H] for u in range(8))
    col_a = _dot(col_ea, w)
    g16 = sum(g_ref[u * H:(u + 1) * H, u * H:(u + 1) * H] for u in range(8))
    m1 = lax.dot_general(w, g16, (((0,), (0,)), ((), ())),
                         precision=_HIGH, preferred_element_type=jnp.float32)
    m2 = _dot(m1, w)
    col_a2 = jnp.sum(m2 * jnp.eye(H, dtype=jnp.float32), axis=0,
                     keepdims=True)
    sea = sea_ref[0:NP] + sea_ref[NP:2 * NP]
    sa = _dot(sea, w)
    outdeg = od_ref[0:NP, 0:1] + od_ref[NP:2 * NP, 0:1]
    p = p_ref[...]
    sum_b = col_a + jnp.sum(outdeg * p, axis=0, keepdims=True)
    sumsq = (col_a2 + jnp.sum(outdeg * p * p, axis=0, keepdims=True)
             + 2.0 * jnp.sum(p * sa, axis=0, keepdims=True))
    mean_b = sum_b / float(E)
    var = sumsq / float(E) - mean_b * mean_b
    a = g1_ref[...] * lax.rsqrt(var + 1e-5)
    c = bt1_ref[...] - mean_b * a
    o_ref[...] = jnp.concatenate([a, c], axis=0)


NPP = NP // 8
NNP = N // 8


def _tile8(v):
    return jnp.concatenate([v] * 8, axis=1)


def _final_body(qp_ref, hp_ref, cp_ref, w2k_ref, b2p_ref, w3k_ref, b3p_ref,
                g2p_ref, bt2p_ref, w4k_ref, b4p_ref, bk_ref, o_ref):
    s = hp_ref[0:NNP] + hp_ref[NPP:NPP + NNP]
    cnt0 = cp_ref[0:NNP] + cp_ref[NPP:NPP + NNP]
    cnt = _dot(cnt0, bk_ref[...])
    sm = s / jnp.maximum(cnt, 1.0)
    agg = (_dot(sm, w2k_ref[...])
           + b2p_ref[...] * (cnt > 0).astype(jnp.float32))
    z1 = qp_ref[...] + _dot(agg, w3k_ref[...]) + b3p_ref[...]
    s128 = jnp.sum(z1, axis=0, keepdims=True)
    q128 = jnp.sum(z1 * z1, axis=0, keepdims=True)
    m16 = sum(s128[0:1, u * H:(u + 1) * H] for u in range(8)) / float(N)
    q16 = sum(q128[0:1, u * H:(u + 1) * H] for u in range(8)) / float(N)
    m = _tile8(m16)
    v = _tile8(q16) - m * m
    zn = jnp.maximum((z1 - m) * lax.rsqrt(v + 1e-5) * g2p_ref[...]
                     + bt2p_ref[...], 0.0)
    o_ref[...] = _dot(zn, w4k_ref[...]) + b4p_ref[...]


_MESH = plsc.VectorSubcoreMesh(core_axis_name="c", subcore_axis_name="s")


@functools.partial(
    pl.kernel,
    mesh=_MESH,
    compiler_params=pltpu.CompilerParams(use_tc_tiling_on_sc=False),
    out_type=[
        jax.ShapeDtypeStruct((NC * NP, H), jnp.float32),
        jax.ShapeDtypeStruct((NC * NP, H), jnp.float32),
    ],
    scratch_types=[
        pltpu.VMEM((NSUB, CB), jnp.int32),
        pltpu.VMEM((NSUB, CB), jnp.int32),
        pltpu.VMEM((CB, H), jnp.float32),
        pltpu.VMEM_SHARED((NP, H), jnp.float32),
        pltpu.VMEM_SHARED((NP, H), jnp.float32),
    ],
)
def _sc_hist(send_hbm, rec_hbm, cnt1_hbm, zeros_hbm, outs_hbm, outr_hbm,
             sidx_v, ridx_v, cbuf, acc_s, acc_r):
    cid = lax.axis_index("c")
    sid = lax.axis_index("s")
    wid = sid * NC + cid

    @pl.when(sid == 0)
    def _():
        pltpu.sync_copy(zeros_hbm, acc_s)
        pltpu.sync_copy(zeros_hbm, acc_r)

    pltpu.sync_copy(cnt1_hbm, cbuf)
    pltpu.sync_copy(send_hbm.at[wid], sidx_v)
    pltpu.sync_copy(rec_hbm.at[wid], ridx_v)
    plsc.subcore_barrier()

    def body(j, carry):
        pltpu.sync_copy(cbuf, acc_s.at[sidx_v.at[j]], add=True)
        pltpu.sync_copy(cbuf, acc_r.at[ridx_v.at[j]], add=True)
        return carry

    lax.fori_loop(0, NSUB, body, 0, unroll=False)
    plsc.subcore_barrier()
    rows = NP // NS
    pltpu.sync_copy(acc_s.at[pl.ds(sid * rows, rows)],
                    outs_hbm.at[pl.ds(cid * NP + sid * rows, rows)])
    pltpu.sync_copy(acc_r.at[pl.ds(sid * rows, rows)],
                    outr_hbm.at[pl.ds(cid * NP + sid * rows, rows)])


@functools.partial(
    pl.kernel,
    mesh=_MESH,
    compiler_params=pltpu.CompilerParams(use_tc_tiling_on_sc=False),
    out_type=jax.ShapeDtypeStruct((NC * NP, H), jnp.float32),
    scratch_types=[
        pltpu.VMEM((NSUB, CB), jnp.int32),
        pltpu.VMEM((BIG, H), jnp.float32),
        pltpu.VMEM_SHARED((NP, H), jnp.float32),
        pltpu.SemaphoreType.DMA,
    ],
)
def _sc_sea(ea_hbm, send_hbm, zeros_hbm, out_hbm, idx_v, abuf, acc, dsem):
    cid = lax.axis_index("c")
    sid = lax.axis_index("s")
    wid = sid * NC + cid
    base = wid * TE

    @pl.when(sid == 0)
    def _():
        pltpu.sync_copy(zeros_hbm, acc)

    pltpu.sync_copy(send_hbm.at[wid], idx_v)
    plsc.subcore_barrier()

    def big_body(b, carry):
        row0 = base + b * BIG
        pltpu.async_copy(ea_hbm.at[pl.ds(row0, BIG)], abuf, dsem).wait()
        for j in range(SPB):
            pltpu.sync_copy(abuf.at[pl.ds(j * CB, CB)],
                            acc.at[idx_v.at[b * SPB + j]], add=True)
        return carry

    lax.fori_loop(0, NBIG, big_body, 0, unroll=False)
    plsc.subcore_barrier()
    rows = NP // NS
    pltpu.sync_copy(acc.at[pl.ds(sid * rows, rows)],
                    out_hbm.at[pl.ds(cid * NP + sid * rows, rows)])


@functools.partial(
    pl.kernel,
    mesh=_MESH,
    compiler_params=pltpu.CompilerParams(use_tc_tiling_on_sc=False),
    out_type=jax.ShapeDtypeStruct((NC * NP, H), jnp.float32),
    scratch_types=[
        pltpu.VMEM((NSUB, CB), jnp.int32),
        pltpu.VMEM((NSUB, CB), jnp.int32),
        pltpu.VMEM((BIG, H), jnp.float32),
        pltpu.VMEM((BIG // 8, 128), jnp.float32),
        pltpu.VMEM((2, H), jnp.float32),
        pltpu.VMEM_SHARED((NP, H), jnp.float32),
        pltpu.SemaphoreType.DMA,
        pltpu.SemaphoreType.DMA,
    ],
)
def _sc_edge(p_hbm, a_hbm, send_hbm, rec_hbm, ac_hbm, zeros_hbm,
             outh_hbm, sidx_v, ridx_v, gbuf, abuf, acv, acc_h, gsem, dsem):
    cid = lax.axis_index("c")
    sid = lax.axis_index("s")
    wid = sid * NC + cid
    basep = wid * (TE // 8)

    @pl.when(sid == 0)
    def _():
        pltpu.sync_copy(zeros_hbm, acc_h)

    pltpu.sync_copy(ac_hbm, acv)
    pltpu.sync_copy(send_hbm.at[wid], sidx_v)
    pltpu.sync_copy(rec_hbm.at[wid], ridx_v)
    plsc.subcore_barrier()

    av = acv[0]
    cv = acv[1]

    def big_body(b, carry):
        rowp = basep + b * (BIG // 8)
        a_cp = pltpu.async_copy(a_hbm.at[pl.ds(rowp, BIG // 8)], abuf, dsem)
        gathers = []
        for j in range(SPB):
            gathers.append(pltpu.async_copy(
                p_hbm.at[sidx_v.at[b * SPB + j]],
                gbuf.at[pl.ds(j * CB, CB)], gsem))
        a_cp.wait()
        for g in gathers:
            g.wait()

        def row_body(r0, carry2):
            for u in range(ROWU):
                r = r0 * ROWU + u
                hv = gbuf[r] + abuf[r0, u * H:(u + 1) * H]
                gbuf[r] = jnp.maximum(hv * av + cv, 0.0)
            return carry2

        lax.fori_loop(0, BIG // ROWU, row_body, 0, unroll=False)
        for j in range(SPB):
            pltpu.sync_copy(gbuf.at[pl.ds(j * CB, CB)],
                            acc_h.at[ridx_v.at[b * SPB + j]], add=True)
        return carry

    lax.fori_loop(0, NBIG, big_body, 0, unroll=False)
    plsc.subcore_barrier()
    rows = NP // NS
    pltpu.sync_copy(acc_h.at[pl.ds(sid * rows, rows)],
                    outh_hbm.at[pl.ds(cid * NP + sid * rows, rows)])


def kernel(x, edge_index, edge_attr, u, batch, W1, b1, g1, bt1, W2, b2,
           W3, b3, g2, bt2, W4, b4):
    del u, batch
    send = edge_index[0].astype(jnp.int32).reshape(NW, NSUB, CB)
    rec = edge_index[1].astype(jnp.int32).reshape(NW, NSUB, CB)
    eap = edge_attr.reshape(EP, 128)
    wd = jnp.kron(jnp.eye(8, dtype=jnp.float32), W1[F:])

    cnt1 = jnp.zeros((CB, H), jnp.float32).at[:, 0].set(1.0)
    zeros16 = jnp.zeros((NP, H), jnp.float32)

    od_acc, cn_acc = _sc_hist(send, rec, cnt1, zeros16)

    p, q = pl.pallas_call(
        _pq_body,
        out_shape=[jax.ShapeDtypeStruct((NP, H), jnp.float32),
                   jax.ShapeDtypeStruct((N, H), jnp.float32)],
    )(x, W1[:F], W3[:F])

    nblk = EP // EPBLK
    a_mat, ea_lin, gacc, csums = pl.pallas_call(
        _edge_lin_body,
        grid=(nblk,),
        in_specs=[pl.BlockSpec((EPBLK, 128), lambda i: (i, 0)),
                  pl.BlockSpec((128, 128), lambda i: (0, 0))],
        out_specs=[pl.BlockSpec((EPBLK, 128), lambda i: (i, 0)),
                   pl.BlockSpec((EPBLK, 128), lambda i: (i, 0)),
                   pl.BlockSpec((128, 128), lambda i: (0, 0)),
                   pl.BlockSpec((8, 128), lambda i: (0, 0))],
        out_shape=[jax.ShapeDtypeStruct((EP, 128), jnp.float32),
                   jax.ShapeDtypeStruct((EP, 128), jnp.float32),
                   jax.ShapeDtypeStruct((128, 128), jnp.float32),
                   jax.ShapeDtypeStruct((8, 128), jnp.float32)],
    )(eap, wd)

    sea_acc = _sc_sea(ea_lin.reshape(E, H), send, zeros16)

    ac = pl.pallas_call(
        _stats_body,
        out_shape=jax.ShapeDtypeStruct((2, H), jnp.float32),
    )(gacc, csums, W1[F:], sea_acc, od_acc, p, g1[None], bt1[None])

    acc_h = _sc_edge(p, a_mat, send, rec, ac, zeros16)

    eye8 = jnp.eye(8, dtype=jnp.float32)
    bsel = jnp.zeros((H, H), jnp.float32).at[0].set(1.0)
    zp = pl.pallas_call(
        _final_body,
        out_shape=jax.ShapeDtypeStruct((NNP, 128), jnp.float32),
    )(q.reshape(NNP, 128), acc_h.reshape(NC * NPP, 128),
      cn_acc.reshape(NC * NPP, 128), jnp.kron(eye8, W2),
      jnp.tile(b2[None], (1, 8)), jnp.kron(eye8, W3[F:]),
      jnp.tile(b3[None], (1, 8)), jnp.tile(g2[None], (1, 8)),
      jnp.tile(bt2[None], (1, 8)), jnp.kron(eye8, W4),
      jnp.tile(b4[None], (1, 8)), jnp.kron(eye8, bsel))
    return zp.reshape(N, H)


# speedup vs baseline: 1.6345x; 1.0665x over previous
import functools

import jax
import jax.numpy as jnp
from jax import lax
from jax.experimental import pallas as pl
from jax.experimental.pallas import tpu as pltpu
from jax.experimental.pallas import tpu_sc as plsc

N = 10000
E = 320000
F = 128
H = 16

NC = 2
NS = 16
NW = NC * NS
TE = E // NW
CB = 125
NSUB = TE // CB
BIG = 2000
NBIG = TE // BIG
SPB = BIG // CB
NP = 10240
ROWU = 8
EP = E // 8
EPBLK = 2000

_HIGH = lax.Precision.HIGHEST


def _dot(a, b):
    return jnp.dot(a, b, precision=_HIGH, preferred_element_type=jnp.float32)


def _pq_body(x_ref, w1a_ref, w3a_ref, p_ref, q_ref):
    xv = x_ref[...]
    p_ref[...] = jnp.concatenate(
        [_dot(xv, w1a_ref[...]), jnp.zeros((NP - N, H), jnp.float32)], axis=0)
    q_ref[...] = _dot(xv, w3a_ref[...])


def _edge_lin_body(ea_ref, w_ref, a_ref, el_ref, g_ref, cs_ref):
    blk = ea_ref[...]
    a_ref[...] = _dot(blk, w_ref[...])
    el_ref[...] = blk
    gblk = lax.dot_general(blk, blk, (((0,), (0,)), ((), ())),
                           precision=_HIGH,
                           preferred_element_type=jnp.float32)
    csblk = jnp.concatenate(
        [jnp.sum(blk, axis=0, keepdims=True),
         jnp.zeros((7, 128), jnp.float32)], axis=0)

    @pl.when(pl.program_id(0) == 0)
    def _():
        g_ref[...] = jnp.zeros((128, 128), jnp.float32)
        cs_ref[...] = jnp.zeros((8, 128), jnp.float32)

    g_ref[...] += gblk
    cs_ref[...] += csblk


def _fold8(v128):
    return sum(v128[0:1, u * H:(u + 1) * H] for u in range(8))


def _stats_body(g_ref, cs_ref, w_ref, wd_ref, bk_ref, sea_ref, od_ref,
                p_ref, g1_ref, bt1_ref, o_ref):
    w = w_ref[...]
    cs = cs_ref[...]
    col_a = _dot(_fold8(cs[0:1]), w)
    g16 = sum(g_ref[u * H:(u + 1) * H, u * H:(u + 1) * H] for u in range(8))
    m1 = lax.dot_general(w, g16, (((0,), (0,)), ((), ())),
                         precision=_HIGH, preferred_element_type=jnp.float32)
    m2 = _dot(m1, w)
    col_a2 = jnp.sum(m2 * jnp.eye(H, dtype=jnp.float32), axis=0,
                     keepdims=True)
    seap = sea_ref[0:NPP] + sea_ref[NPP:2 * NPP]
    sap = _dot(seap, wd_ref[...])
    odp = od_ref[0:NPP] + od_ref[NPP:2 * NPP]
    odb = _dot(odp, bk_ref[...])
    pp = p_ref[...]
    sum_b = col_a + _fold8(jnp.sum(odb * pp, axis=0, keepdims=True))
    sumsq = (col_a2
             + _fold8(jnp.sum(odb * pp * pp, axis=0, keepdims=True))
             + 2.0 * _fold8(jnp.sum(pp * sap, axis=0, keepdims=True)))
    mean_b = sum_b / float(E)
    var = sumsq / float(E) - mean_b * mean_b
    a = g1_ref[...] * lax.rsqrt(var + 1e-5)
    c = bt1_ref[...] - mean_b * a
    o_ref[...] = jnp.concatenate([a, c], axis=0)


NPP = NP // 8
NNP = N // 8


def _tile8(v):
    return jnp.concatenate([v] * 8, axis=1)


def _final_body(qp_ref, hp_ref, cp_ref, w2k_ref, b2p_ref, w3k_ref, b3p_ref,
                g2p_ref, bt2p_ref, w4k_ref, b4p_ref, bk_ref, o_ref):
    s = hp_ref[0:NNP] + hp_ref[NPP:NPP + NNP]
    cnt0 = cp_ref[0:NNP] + cp_ref[NPP:NPP + NNP]
    cnt = _dot(cnt0, bk_ref[...])
    sm = s / jnp.maximum(cnt, 1.0)
    agg = (_dot(sm, w2k_ref[...])
           + b2p_ref[...] * (cnt > 0).astype(jnp.float32))
    z1 = qp_ref[...] + _dot(agg, w3k_ref[...]) + b3p_ref[...]
    s128 = jnp.sum(z1, axis=0, keepdims=True)
    q128 = jnp.sum(z1 * z1, axis=0, keepdims=True)
    m16 = sum(s128[0:1, u * H:(u + 1) * H] for u in range(8)) / float(N)
    q16 = sum(q128[0:1, u * H:(u + 1) * H] for u in range(8)) / float(N)
    m = _tile8(m16)
    v = _tile8(q16) - m * m
    zn = jnp.maximum((z1 - m) * lax.rsqrt(v + 1e-5) * g2p_ref[...]
                     + bt2p_ref[...], 0.0)
    o_ref[...] = _dot(zn, w4k_ref[...]) + b4p_ref[...]


_MESH = plsc.VectorSubcoreMesh(core_axis_name="c", subcore_axis_name="s")


@functools.partial(
    pl.kernel,
    mesh=_MESH,
    compiler_params=pltpu.CompilerParams(use_tc_tiling_on_sc=False),
    out_type=[
        jax.ShapeDtypeStruct((NC * NP, H), jnp.float32),
        jax.ShapeDtypeStruct((NC * NP, H), jnp.float32),
    ],
    scratch_types=[
        pltpu.VMEM((NSUB, CB), jnp.int32),
        pltpu.VMEM((NSUB, CB), jnp.int32),
        pltpu.VMEM((CB, H), jnp.float32),
        pltpu.VMEM_SHARED((NP, H), jnp.float32),
        pltpu.VMEM_SHARED((NP, H), jnp.float32),
    ],
)
def _sc_hist(send_hbm, rec_hbm, cnt1_hbm, zeros_hbm, outs_hbm, outr_hbm,
             sidx_v, ridx_v, cbuf, acc_s, acc_r):
    cid = lax.axis_index("c")
    sid = lax.axis_index("s")
    wid = sid * NC + cid

    @pl.when(sid == 0)
    def _():
        pltpu.sync_copy(zeros_hbm, acc_s)
        pltpu.sync_copy(zeros_hbm, acc_r)

    pltpu.sync_copy(cnt1_hbm, cbuf)
    pltpu.sync_copy(send_hbm.at[wid], sidx_v)
    pltpu.sync_copy(rec_hbm.at[wid], ridx_v)
    plsc.subcore_barrier()

    def body(j, carry):
        pltpu.sync_copy(cbuf, acc_s.at[sidx_v.at[j]], add=True)
        pltpu.sync_copy(cbuf, acc_r.at[ridx_v.at[j]], add=True)
        return carry

    lax.fori_loop(0, NSUB, body, 0, unroll=False)
    plsc.subcore_barrier()
    rows = NP // NS
    pltpu.sync_copy(acc_s.at[pl.ds(sid * rows, rows)],
                    outs_hbm.at[pl.ds(cid * NP + sid * rows, rows)])
    pltpu.sync_copy(acc_r.at[pl.ds(sid * rows, rows)],
                    outr_hbm.at[pl.ds(cid * NP + sid * rows, rows)])


@functools.partial(
    pl.kernel,
    mesh=_MESH,
    compiler_params=pltpu.CompilerParams(use_tc_tiling_on_sc=False),
    out_type=jax.ShapeDtypeStruct((NC * NP, H), jnp.float32),
    scratch_types=[
        pltpu.VMEM((NSUB, CB), jnp.int32),
        pltpu.VMEM((BIG, H), jnp.float32),
        pltpu.VMEM_SHARED((NP, H), jnp.float32),
        pltpu.SemaphoreType.DMA,
    ],
)
def _sc_sea(ea_hbm, send_hbm, zeros_hbm, out_hbm, idx_v, abuf, acc, dsem):
    cid = lax.axis_index("c")
    sid = lax.axis_index("s")
    wid = sid * NC + cid
    base = wid * TE

    @pl.when(sid == 0)
    def _():
        pltpu.sync_copy(zeros_hbm, acc)

    pltpu.sync_copy(send_hbm.at[wid], idx_v)
    plsc.subcore_barrier()

    def big_body(b, carry):
        row0 = base + b * BIG
        pltpu.async_copy(ea_hbm.at[pl.ds(row0, BIG)], abuf, dsem).wait()
        for j in range(SPB):
            pltpu.sync_copy(abuf.at[pl.ds(j * CB, CB)],
                            acc.at[idx_v.at[b * SPB + j]], add=True)
        return carry

    lax.fori_loop(0, NBIG, big_body, 0, unroll=False)
    plsc.subcore_barrier()
    rows = NP // NS
    pltpu.sync_copy(acc.at[pl.ds(sid * rows, rows)],
                    out_hbm.at[pl.ds(cid * NP + sid * rows, rows)])


@functools.partial(
    pl.kernel,
    mesh=_MESH,
    compiler_params=pltpu.CompilerParams(use_tc_tiling_on_sc=False),
    out_type=jax.ShapeDtypeStruct((NC * NP, H), jnp.float32),
    scratch_types=[
        pltpu.VMEM((NSUB, CB), jnp.int32),
        pltpu.VMEM((NSUB, CB), jnp.int32),
        pltpu.VMEM((BIG, H), jnp.float32),
        pltpu.VMEM((BIG // 8, 128), jnp.float32),
        pltpu.VMEM((2, H), jnp.float32),
        pltpu.VMEM_SHARED((NP, H), jnp.float32),
        pltpu.SemaphoreType.DMA,
        pltpu.SemaphoreType.DMA,
    ],
)
def _sc_edge(p_hbm, a_hbm, send_hbm, rec_hbm, ac_hbm, zeros_hbm,
             outh_hbm, sidx_v, ridx_v, gbuf, abuf, acv, acc_h, gsem, dsem):
    cid = lax.axis_index("c")
    sid = lax.axis_index("s")
    wid = sid * NC + cid
    basep = wid * (TE // 8)

    @pl.when(sid == 0)
    def _():
        pltpu.sync_copy(zeros_hbm, acc_h)

    pltpu.sync_copy(ac_hbm, acv)
    pltpu.sync_copy(send_hbm.at[wid], sidx_v)
    pltpu.sync_copy(rec_hbm.at[wid], ridx_v)
    plsc.subcore_barrier()

    av = acv[0]
    cv = acv[1]

    def big_body(b, carry):
        rowp = basep + b * (BIG // 8)
        a_cp = pltpu.async_copy(a_hbm.at[pl.ds(rowp, BIG // 8)], abuf, dsem)
        gathers = []
        for j in range(SPB):
            gathers.append(pltpu.async_copy(
                p_hbm.at[sidx_v.at[b * SPB + j]],
                gbuf.at[pl.ds(j * CB, CB)], gsem))
        a_cp.wait()
        for g in gathers:
            g.wait()

        def row_body(r0, carry2):
            for u in range(ROWU):
                r = r0 * ROWU + u
                hv = gbuf[r] + abuf[r0, u * H:(u + 1) * H]
                gbuf[r] = jnp.maximum(hv * av + cv, 0.0)
            return carry2

        lax.fori_loop(0, BIG // ROWU, row_body, 0, unroll=False)
        for j in range(SPB):
            pltpu.sync_copy(gbuf.at[pl.ds(j * CB, CB)],
                            acc_h.at[ridx_v.at[b * SPB + j]], add=True)
        return carry

    lax.fori_loop(0, NBIG, big_body, 0, unroll=False)
    plsc.subcore_barrier()
    rows = NP // NS
    pltpu.sync_copy(acc_h.at[pl.ds(sid * rows, rows)],
                    outh_hbm.at[pl.ds(cid * NP + sid * rows, rows)])


def kernel(x, edge_index, edge_attr, u, batch, W1, b1, g1, bt1, W2, b2,
           W3, b3, g2, bt2, W4, b4):
    del u, batch
    send = edge_index[0].astype(jnp.int32).reshape(NW, NSUB, CB)
    rec = edge_index[1].astype(jnp.int32).reshape(NW, NSUB, CB)
    eap = edge_attr.reshape(EP, 128)
    wd = jnp.kron(jnp.eye(8, dtype=jnp.float32), W1[F:])

    cnt1 = jnp.zeros((CB, H), jnp.float32).at[:, 0].set(1.0)
    zeros16 = jnp.zeros((NP, H), jnp.float32)

    od_acc, cn_acc = _sc_hist(send, rec, cnt1, zeros16)

    p, q = pl.pallas_call(
        _pq_body,
        out_shape=[jax.ShapeDtypeStruct((NP, H), jnp.float32),
                   jax.ShapeDtypeStruct((N, H), jnp.float32)],
    )(x, W1[:F], W3[:F])

    nblk = EP // EPBLK
    a_mat, ea_lin, gacc, csums = pl.pallas_call(
        _edge_lin_body,
        grid=(nblk,),
        in_specs=[pl.BlockSpec((EPBLK, 128), lambda i: (i, 0)),
                  pl.BlockSpec((128, 128), lambda i: (0, 0))],
        out_specs=[pl.BlockSpec((EPBLK, 128), lambda i: (i, 0)),
                   pl.BlockSpec((EPBLK, 128), lambda i: (i, 0)),
                   pl.BlockSpec((128, 128), lambda i: (0, 0)),
                   pl.BlockSpec((8, 128), lambda i: (0, 0))],
        out_shape=[jax.ShapeDtypeStruct((EP, 128), jnp.float32),
                   jax.ShapeDtypeStruct((EP, 128), jnp.float32),
                   jax.ShapeDtypeStruct((128, 128), jnp.float32),
                   jax.ShapeDtypeStruct((8, 128), jnp.float32)],
    )(eap, wd)

    sea_acc = _sc_sea(ea_lin.reshape(E, H), send, zeros16)

    eye8 = jnp.eye(8, dtype=jnp.float32)
    bsel = jnp.zeros((H, H), jnp.float32).at[0].set(1.0)
    bk = jnp.kron(eye8, bsel)

    ac = pl.pallas_call(
        _stats_body,
        out_shape=jax.ShapeDtypeStruct((2, H), jnp.float32),
    )(gacc, csums, W1[F:], wd, bk, sea_acc.reshape(NC * NPP, 128),
      od_acc.reshape(NC * NPP, 128), p.reshape(NPP, 128),
      g1[None], bt1[None])

    acc_h = _sc_edge(p, a_mat, send, rec, ac, zeros16)
    zp = pl.pallas_call(
        _final_body,
        out_shape=jax.ShapeDtypeStruct((NNP, 128), jnp.float32),
    )(q.reshape(NNP, 128), acc_h.reshape(NC * NPP, 128),
      cn_acc.reshape(NC * NPP, 128), jnp.kron(eye8, W2),
      jnp.tile(b2[None], (1, 8)), jnp.kron(eye8, W3[F:]),
      jnp.tile(b3[None], (1, 8)), jnp.tile(g2[None], (1, 8)),
      jnp.tile(bt2[None], (1, 8)), jnp.kron(eye8, W4),
      jnp.tile(b4[None], (1, 8)), jnp.kron(eye8, bsel))
    return zp.reshape(N, H)
